# Initial kernel scaffold; baseline (speedup 1.0000x reference)
import jax
import jax.numpy as jnp
from jax.experimental import pallas as pl


def kernel(basis, outcome, basis_table, outcome_table):
    raise NotImplementedError("write your pallas kernel here")


# speedup vs baseline: 3.6614x; 3.6614x over previous
import functools

import jax
import jax.numpy as jnp
from jax import lax
from jax.experimental import pallas as pl
from jax.experimental.pallas import tpu as pltpu
from jax.experimental.pallas import tpu_sc as plsc

EMBED = 64
NC, NS = 2, 16
NW = NC * NS
LANES = 16
PAIRS = 512
SUB = 128


def _sc_expand(pair_table, b_even, b_odd, o_even, o_odd, n_pairs):
    pairs_per_w = n_pairs // NW
    n_chunks = pairs_per_w // PAIRS
    mesh = plsc.VectorSubcoreMesh(core_axis_name="c", subcore_axis_name="s",
                                  num_cores=NC, num_subcores=NS)

    @functools.partial(
        pl.kernel,
        out_type=jax.ShapeDtypeStruct((n_pairs, 2 * EMBED), jnp.float32),
        mesh=mesh,
        scratch_types=[
            pltpu.VMEM((PAIRS,), jnp.int32),
            pltpu.VMEM((PAIRS,), jnp.int32),
            pltpu.VMEM((PAIRS,), jnp.int32),
            pltpu.VMEM((PAIRS,), jnp.int32),
            pltpu.VMEM((PAIRS // SUB, SUB), jnp.int32),
            pltpu.VMEM((PAIRS, 2 * EMBED), jnp.float32),
            pltpu.SemaphoreType.DMA,
        ],
    )
    def k(table_hbm, be_hbm, bo_hbm, oe_hbm, oo_hbm, out_hbm,
          be_v, bo_v, oe_v, oo_v, pair_v, rows_v, sem):
        wid = lax.axis_index("s") * NC + lax.axis_index("c")
        base_pair = wid * pairs_per_w

        def chunk_body(g, carry):
            q0 = pl.multiple_of(base_pair + g * PAIRS, PAIRS)
            pltpu.sync_copy(be_hbm.at[pl.ds(q0, PAIRS)], be_v)
            pltpu.sync_copy(bo_hbm.at[pl.ds(q0, PAIRS)], bo_v)
            pltpu.sync_copy(oe_hbm.at[pl.ds(q0, PAIRS)], oe_v)
            pltpu.sync_copy(oo_hbm.at[pl.ds(q0, PAIRS)], oo_v)
            for r in range(PAIRS // SUB):
                def col_body(c, c_carry):
                    off = (r * (SUB // LANES) + c) * LANES
                    sl = pl.ds(off, LANES)
                    pair_v[r, pl.ds(off - r * SUB, LANES)] = (
                        (be_v[sl] * 2 + oe_v[sl]) * 6
                        + bo_v[sl] * 2 + oo_v[sl])
                    return c_carry
                lax.fori_loop(0, SUB // LANES, col_body, 0)
            copies = [
                pltpu.async_copy(table_hbm.at[pair_v.at[r]],
                                 rows_v.at[pl.ds(r * SUB, SUB)], sem)
                for r in range(PAIRS // SUB)
            ]
            for cp in copies:
                cp.wait()
            pltpu.sync_copy(rows_v, out_hbm.at[pl.ds(q0, PAIRS)])
            return carry

        lax.fori_loop(0, n_chunks, chunk_body, 0)

    return k(pair_table, b_even, b_odd, o_even, o_odd)


def kernel(basis, outcome, basis_table, outcome_table):
    n_rows = basis.shape[0] * basis.shape[1]
    n_pairs = n_rows // 2
    n_comb = basis_table.shape[0] * outcome_table.shape[0]
    comb = (basis_table[:, None, :] + outcome_table[None, :, :]).reshape(
        n_comb, EMBED)
    pair_table = jnp.concatenate(
        [jnp.repeat(comb, n_comb, axis=0), jnp.tile(comb, (n_comb, 1))], axis=1)
    b2 = basis.reshape(n_pairs, 2).astype(jnp.int32)
    o2 = outcome.reshape(n_pairs, 2).astype(jnp.int32)
    out = _sc_expand(pair_table, b2[:, 0], b2[:, 1], o2[:, 0], o2[:, 1],
                     n_pairs)
    return out.reshape(basis.shape[0], basis.shape[1], EMBED)


# speedup vs baseline: 4.1473x; 1.1327x over previous
import functools

import jax
import jax.numpy as jnp
from jax import lax
from jax.experimental import pallas as pl
from jax.experimental.pallas import tpu as pltpu
from jax.experimental.pallas import tpu_sc as plsc

EMBED = 64
NC, NS = 2, 16
NW = NC * NS
LANES = 16
PAIRS = 256
SUB = 128


def _sc_expand(pair_table, basis16, outcome16, n_pairs):
    pairs_per_w = n_pairs // NW
    n_chunks = pairs_per_w // PAIRS
    assert n_chunks % 2 == 0
    mesh = plsc.VectorSubcoreMesh(core_axis_name="c", subcore_axis_name="s",
                                  num_cores=NC, num_subcores=NS)

    @functools.partial(
        pl.kernel,
        out_type=jax.ShapeDtypeStruct((n_pairs, 2 * EMBED), jnp.float32),
        mesh=mesh,
        scratch_types=[
            pltpu.VMEM((PAIRS,), jnp.int32),
            pltpu.VMEM((PAIRS,), jnp.int32),
            pltpu.VMEM((PAIRS // SUB, SUB), jnp.int32),
            pltpu.VMEM((PAIRS, 2 * EMBED), jnp.float32),
            pltpu.VMEM((PAIRS, 2 * EMBED), jnp.float32),
            pltpu.SemaphoreType.DMA,
            pltpu.SemaphoreType.DMA,
            pltpu.SemaphoreType.DMA,
            pltpu.SemaphoreType.DMA,
        ],
    )
    def k(table_hbm, basis_hbm, outcome_hbm, out_hbm,
          bas_v, ocm_v, pair_v, rows_a, rows_b, sem_i, sem_g, sem_oa, sem_ob):
        wid = lax.axis_index("s") * NC + lax.axis_index("c")
        base_pair = wid * pairs_per_w

        def start_idx(g):
            g_safe = jnp.where(g < n_chunks, g, 0)
            q0 = pl.multiple_of(base_pair + g_safe * PAIRS, PAIRS)
            pltpu.async_copy(basis_hbm.at[pl.ds(q0, PAIRS)], bas_v, sem_i)
            pltpu.async_copy(outcome_hbm.at[pl.ds(q0, PAIRS)], ocm_v, sem_i)

        def wait_idx(q0):
            pltpu.make_async_copy(basis_hbm.at[pl.ds(q0, PAIRS)],
                                  bas_v, sem_i).wait()
            pltpu.make_async_copy(outcome_hbm.at[pl.ds(q0, PAIRS)],
                                  ocm_v, sem_i).wait()

        def chunk(g, rows_ref, sem_o, pre_wait_out):
            q0 = pl.multiple_of(base_pair + g * PAIRS, PAIRS)
            wait_idx(q0)
            for r in range(PAIRS // SUB):
                def col_body(c, carry):
                    off = pl.multiple_of((r * (SUB // LANES) + c) * LANES,
                                         LANES)
                    x = bas_v[pl.ds(off, LANES)]
                    y = ocm_v[pl.ds(off, LANES)]
                    z = x * 2 + y
                    pair_v[r, pl.ds(c * LANES, LANES)] = (
                        (z & 0xFFFF) * 6 + (z >> 16))
                    return carry
                lax.fori_loop(0, SUB // LANES, col_body, 0)
            start_idx(g + 1)
            if pre_wait_out:
                pltpu.make_async_copy(
                    rows_ref, out_hbm.at[pl.ds(q0, PAIRS)], sem_o).wait()
            gathers = [
                pltpu.async_copy(table_hbm.at[pair_v.at[r]],
                                 rows_ref.at[pl.ds(r * SUB, SUB)], sem_g)
                for r in range(PAIRS // SUB)
            ]
            for cp in gathers:
                cp.wait()
            pltpu.async_copy(rows_ref, out_hbm.at[pl.ds(q0, PAIRS)], sem_o)

        start_idx(0)
        chunk(0, rows_a, sem_oa, False)
        chunk(1, rows_b, sem_ob, False)

        def loop_body(i, carry):
            chunk(2 * i, rows_a, sem_oa, True)
            chunk(2 * i + 1, rows_b, sem_ob, True)
            return carry

        lax.fori_loop(1, n_chunks // 2, loop_body, 0)

        q_last = pl.multiple_of(base_pair, PAIRS)
        wait_idx(q_last)
        pltpu.make_async_copy(rows_a, out_hbm.at[pl.ds(q_last, PAIRS)],
                              sem_oa).wait()
        pltpu.make_async_copy(rows_b, out_hbm.at[pl.ds(q_last, PAIRS)],
                              sem_ob).wait()

    return k(pair_table, basis16, outcome16)


def kernel(basis, outcome, basis_table, outcome_table):
    n_rows = basis.shape[0] * basis.shape[1]
    n_pairs = n_rows // 2
    n_comb = basis_table.shape[0] * outcome_table.shape[0]
    comb = (basis_table[:, None, :] + outcome_table[None, :, :]).reshape(
        n_comb, EMBED)
    pair_table = jnp.concatenate(
        [jnp.repeat(comb, n_comb, axis=0), jnp.tile(comb, (n_comb, 1))], axis=1)
    basis_p = jax.lax.bitcast_convert_type(
        basis.reshape(n_pairs, 2).astype(jnp.int16), jnp.int32)
    outcome_p = jax.lax.bitcast_convert_type(
        outcome.reshape(n_pairs, 2).astype(jnp.int16), jnp.int32)
    out = _sc_expand(pair_table, basis_p, outcome_p, n_pairs)
    return out.reshape(basis.shape[0], basis.shape[1], EMBED)


# speedup vs baseline: 7.2598x; 1.7505x over previous
import functools

import jax
import jax.numpy as jnp
from jax import lax
from jax.experimental import pallas as pl
from jax.experimental.pallas import tpu as pltpu
from jax.experimental.pallas import tpu_sc as plsc

EMBED = 64
NC, NS = 2, 16
NW = NC * NS
LANES = 16
PAIRS = 256
SUB = 128


def _sc_expand(pair_table, basis16, outcome16, n_pairs):
    pairs_per_w = n_pairs // NW
    n_chunks = pairs_per_w // PAIRS
    assert n_chunks % 2 == 0
    mesh = plsc.VectorSubcoreMesh(core_axis_name="c", subcore_axis_name="s",
                                  num_cores=NC, num_subcores=NS)

    @functools.partial(
        pl.kernel,
        out_type=jax.ShapeDtypeStruct((n_pairs, 2 * EMBED), jnp.float32),
        mesh=mesh,
        scratch_types=[
            pltpu.VMEM((PAIRS,), jnp.int32),
            pltpu.VMEM((PAIRS,), jnp.int32),
            pltpu.VMEM((PAIRS // SUB, SUB), jnp.int32),
            pltpu.VMEM((PAIRS, 2 * EMBED), jnp.float32),
            pltpu.VMEM((PAIRS, 2 * EMBED), jnp.float32),
            pltpu.VMEM_SHARED((36, 2 * EMBED), jnp.float32),
            pltpu.SemaphoreType.DMA,
            pltpu.SemaphoreType.DMA,
            pltpu.SemaphoreType.DMA,
            pltpu.SemaphoreType.DMA,
        ],
    )
    def k(table_hbm, basis_hbm, outcome_hbm, out_hbm,
          bas_v, ocm_v, pair_v, rows_a, rows_b, table_sh,
          sem_i, sem_g, sem_oa, sem_ob):
        wid = lax.axis_index("s") * NC + lax.axis_index("c")
        base_pair = wid * pairs_per_w

        @pl.when(lax.axis_index("s") == 0)
        def _():
            pltpu.sync_copy(table_hbm, table_sh)
        plsc.subcore_barrier()

        def start_idx(g):
            g_safe = jnp.where(g < n_chunks, g, 0)
            q0 = pl.multiple_of(base_pair + g_safe * PAIRS, PAIRS)
            pltpu.async_copy(basis_hbm.at[pl.ds(q0, PAIRS)], bas_v, sem_i)
            pltpu.async_copy(outcome_hbm.at[pl.ds(q0, PAIRS)], ocm_v, sem_i)

        def wait_idx(q0):
            pltpu.make_async_copy(basis_hbm.at[pl.ds(q0, PAIRS)],
                                  bas_v, sem_i).wait()
            pltpu.make_async_copy(outcome_hbm.at[pl.ds(q0, PAIRS)],
                                  ocm_v, sem_i).wait()

        def chunk(g, rows_ref, sem_o, pre_wait_out):
            q0 = pl.multiple_of(base_pair + g * PAIRS, PAIRS)
            wait_idx(q0)
            for r in range(PAIRS // SUB):
                def col_body(c, carry):
                    off = pl.multiple_of((r * (SUB // LANES) + c) * LANES,
                                         LANES)
                    x = bas_v[pl.ds(off, LANES)]
                    y = ocm_v[pl.ds(off, LANES)]
                    z = x * 2 + y
                    pair_v[r, pl.ds(c * LANES, LANES)] = (
                        (z & 0xFFFF) * 6 + (z >> 16))
                    return carry
                lax.fori_loop(0, SUB // LANES, col_body, 0)
            start_idx(g + 1)
            if pre_wait_out:
                pltpu.make_async_copy(
                    rows_ref, out_hbm.at[pl.ds(q0, PAIRS)], sem_o).wait()
            gathers = [
                pltpu.async_copy(table_sh.at[pair_v.at[r]],
                                 rows_ref.at[pl.ds(r * SUB, SUB)], sem_g)
                for r in range(PAIRS // SUB)
            ]
            for cp in gathers:
                cp.wait()
            pltpu.async_copy(rows_ref, out_hbm.at[pl.ds(q0, PAIRS)], sem_o)

        start_idx(0)
        chunk(0, rows_a, sem_oa, False)
        chunk(1, rows_b, sem_ob, False)

        def loop_body(i, carry):
            chunk(2 * i, rows_a, sem_oa, True)
            chunk(2 * i + 1, rows_b, sem_ob, True)
            return carry

        lax.fori_loop(1, n_chunks // 2, loop_body, 0)

        q_last = pl.multiple_of(base_pair, PAIRS)
        wait_idx(q_last)
        pltpu.make_async_copy(rows_a, out_hbm.at[pl.ds(q_last, PAIRS)],
                              sem_oa).wait()
        pltpu.make_async_copy(rows_b, out_hbm.at[pl.ds(q_last, PAIRS)],
                              sem_ob).wait()

    return k(pair_table, basis16, outcome16)


def kernel(basis, outcome, basis_table, outcome_table):
    n_rows = basis.shape[0] * basis.shape[1]
    n_pairs = n_rows // 2
    n_comb = basis_table.shape[0] * outcome_table.shape[0]
    comb = (basis_table[:, None, :] + outcome_table[None, :, :]).reshape(
        n_comb, EMBED)
    pair_table = jnp.concatenate(
        [jnp.repeat(comb, n_comb, axis=0), jnp.tile(comb, (n_comb, 1))], axis=1)
    basis_p = jax.lax.bitcast_convert_type(
        basis.reshape(n_pairs, 2).astype(jnp.int16), jnp.int32)
    outcome_p = jax.lax.bitcast_convert_type(
        outcome.reshape(n_pairs, 2).astype(jnp.int16), jnp.int32)
    out = _sc_expand(pair_table, basis_p, outcome_p, n_pairs)
    return out.reshape(basis.shape[0], basis.shape[1], EMBED)


# speedup vs baseline: 11.0349x; 1.5200x over previous
import functools

import jax
import jax.numpy as jnp
from jax import lax
from jax.experimental import pallas as pl
from jax.experimental.pallas import tpu as pltpu
from jax.experimental.pallas import tpu_sc as plsc

EMBED = 64
NC, NS = 2, 16
NW = NC * NS
LANES = 16
PAIRS = 256
SUB = 128


def _shuf(v, idx):
    dnums = lax.GatherDimensionNumbers(
        offset_dims=(), collapsed_slice_dims=(0,), start_index_map=(0,))
    return lax.gather(v, idx[:, None], dnums, (1,),
                      mode=lax.GatherScatterMode.PROMISE_IN_BOUNDS)


def _sc_expand(pair_table, basis16, outcome16, n_pairs):
    pairs_per_w = n_pairs // NW
    n_chunks = pairs_per_w // PAIRS
    assert n_chunks % 2 == 0
    mesh = plsc.VectorSubcoreMesh(core_axis_name="c", subcore_axis_name="s",
                                  num_cores=NC, num_subcores=NS)

    @functools.partial(
        pl.kernel,
        out_type=jax.ShapeDtypeStruct((n_pairs, 2 * EMBED), jnp.float32),
        mesh=mesh,
        scratch_types=[
            pltpu.VMEM((2 * PAIRS,), jnp.int32),
            pltpu.VMEM((2 * PAIRS,), jnp.int32),
            pltpu.VMEM((PAIRS // SUB, SUB), jnp.int32),
            pltpu.VMEM((PAIRS, 2 * EMBED), jnp.float32),
            pltpu.VMEM((PAIRS, 2 * EMBED), jnp.float32),
            pltpu.VMEM_SHARED((36, 2 * EMBED), jnp.float32),
            pltpu.SemaphoreType.DMA,
            pltpu.SemaphoreType.DMA,
            pltpu.SemaphoreType.DMA,
            pltpu.SemaphoreType.DMA,
        ],
    )
    def k(table_hbm, basis_hbm, outcome_hbm, out_hbm,
          bas_v, ocm_v, pair_v, rows_a, rows_b, table_sh,
          sem_i, sem_g, sem_oa, sem_ob):
        wid = lax.axis_index("s") * NC + lax.axis_index("c")
        base_pair = wid * pairs_per_w
        iota = lax.iota(jnp.int32, LANES)
        idx_e = (iota * 2) & (LANES - 1)
        idx_o = idx_e + 1
        lo_half = iota < (LANES // 2)

        @pl.when(lax.axis_index("s") == 0)
        def _():
            pltpu.sync_copy(table_hbm, table_sh)
        plsc.subcore_barrier()

        def start_idx(g):
            g_safe = jnp.where(g < n_chunks, g, 0)
            r0 = pl.multiple_of((base_pair + g_safe * PAIRS) * 2, 2 * PAIRS)
            pltpu.async_copy(basis_hbm.at[pl.ds(r0, 2 * PAIRS)], bas_v, sem_i)
            pltpu.async_copy(outcome_hbm.at[pl.ds(r0, 2 * PAIRS)], ocm_v,
                             sem_i)

        def wait_idx(q0):
            r0 = pl.multiple_of(q0 * 2, 2 * PAIRS)
            pltpu.make_async_copy(basis_hbm.at[pl.ds(r0, 2 * PAIRS)],
                                  bas_v, sem_i).wait()
            pltpu.make_async_copy(outcome_hbm.at[pl.ds(r0, 2 * PAIRS)],
                                  ocm_v, sem_i).wait()

        def chunk(g, rows_ref, sem_o, pre_wait_out):
            q0 = pl.multiple_of(base_pair + g * PAIRS, PAIRS)
            wait_idx(q0)
            for r in range(PAIRS // SUB):
                def col_body(c, carry):
                    off = pl.multiple_of(
                        (r * (SUB // LANES) + c) * 2 * LANES, LANES)
                    x0 = bas_v[pl.ds(off, LANES)]
                    x1 = bas_v[pl.ds(off + LANES, LANES)]
                    y0 = ocm_v[pl.ds(off, LANES)]
                    y1 = ocm_v[pl.ds(off + LANES, LANES)]
                    c0 = x0 * 2 + y0
                    c1 = x1 * 2 + y1
                    ce = jnp.where(lo_half, _shuf(c0, idx_e), _shuf(c1, idx_e))
                    co = jnp.where(lo_half, _shuf(c0, idx_o), _shuf(c1, idx_o))
                    pair_v[r, pl.ds(c * LANES, LANES)] = ce * 6 + co
                    return carry
                lax.fori_loop(0, SUB // LANES, col_body, 0)
            start_idx(g + 1)
            if pre_wait_out:
                pltpu.make_async_copy(
                    rows_ref, out_hbm.at[pl.ds(q0, PAIRS)], sem_o).wait()
            gathers = [
                pltpu.async_copy(table_sh.at[pair_v.at[r]],
                                 rows_ref.at[pl.ds(r * SUB, SUB)], sem_g)
                for r in range(PAIRS // SUB)
            ]
            for cp in gathers:
                cp.wait()
            pltpu.async_copy(rows_ref, out_hbm.at[pl.ds(q0, PAIRS)], sem_o)

        start_idx(0)
        chunk(0, rows_a, sem_oa, False)
        chunk(1, rows_b, sem_ob, False)

        def loop_body(i, carry):
            chunk(2 * i, rows_a, sem_oa, True)
            chunk(2 * i + 1, rows_b, sem_ob, True)
            return carry

        lax.fori_loop(1, n_chunks // 2, loop_body, 0)

        q_last = pl.multiple_of(base_pair, PAIRS)
        wait_idx(q_last)
        pltpu.make_async_copy(rows_a, out_hbm.at[pl.ds(q_last, PAIRS)],
                              sem_oa).wait()
        pltpu.make_async_copy(rows_b, out_hbm.at[pl.ds(q_last, PAIRS)],
                              sem_ob).wait()

    return k(pair_table, basis16, outcome16)


def kernel(basis, outcome, basis_table, outcome_table):
    n_rows = basis.shape[0] * basis.shape[1]
    n_pairs = n_rows // 2
    n_comb = basis_table.shape[0] * outcome_table.shape[0]
    comb = (basis_table[:, None, :] + outcome_table[None, :, :]).reshape(
        n_comb, EMBED)
    pair_table = jnp.concatenate(
        [jnp.repeat(comb, n_comb, axis=0), jnp.tile(comb, (n_comb, 1))], axis=1)
    basis_flat = basis.reshape(n_rows).astype(jnp.int32)
    outcome_flat = outcome.reshape(n_rows).astype(jnp.int32)
    out = _sc_expand(pair_table, basis_flat, outcome_flat, n_pairs)
    return out.reshape(basis.shape[0], basis.shape[1], EMBED)
